# Initial kernel scaffold; baseline (speedup 1.0000x reference)
import jax
import jax.numpy as jnp
from jax.experimental import pallas as pl


def kernel(x, protos, log_alpha, log_sigma):
    raise NotImplementedError("write your pallas kernel here")


# speedup vs baseline: 2.9693x; 2.9693x over previous
import math

import jax
import jax.numpy as jnp
from jax.experimental import pallas as pl
from jax.experimental.pallas import tpu as pltpu

K = 16
DECAY = 0.95
SQRT_2_OVER_PI = math.sqrt(2.0 / math.pi)


def _gelu(x):
    return 0.5 * x * (1.0 + jnp.tanh(SQRT_2_OVER_PI * (x + 0.044715 * x * x * x)))


def _row_normalize(v):
    n = jnp.sqrt(jnp.sum(v * v, axis=-1, keepdims=True))
    return v / jnp.maximum(n, 1e-12)


def _pass1_kernel(x_ref, protos_ref, la_ref, ls_ref,
                  protos_out_ref, gate_ref,
                  sums_ref, counts_ref, nov_ref,
                  *, nblocks, n_tokens):
    i = pl.program_id(0)

    @pl.when(i == 0)
    def _init():
        sums_ref[...] = jnp.zeros_like(sums_ref)
        counts_ref[...] = jnp.zeros_like(counts_ref)
        nov_ref[0, 0] = 0.0

    x = x_ref[...]
    p = protos_ref[...]
    pn = _row_normalize(p)
    inv_norm = 1.0 / jnp.maximum(
        jnp.sqrt(jnp.sum(x * x, axis=1, keepdims=True)), 1e-12)
    sim = jax.lax.dot_general(
        x, pn, (((1,), (1,)), ((), ())),
        preferred_element_type=jnp.float32) * inv_norm
    m = jnp.max(sim, axis=1, keepdims=True)
    iota = jax.lax.broadcasted_iota(jnp.int32, sim.shape, 1)
    idx = jnp.min(jnp.where(sim == m, iota, K), axis=1, keepdims=True)
    one_hot = (iota == idx).astype(jnp.float32)
    sums_ref[...] += jax.lax.dot_general(
        one_hot, x, (((0,), (0,)), ((), ())),
        preferred_element_type=jnp.float32)
    counts_ref[...] += jax.lax.dot_general(
        one_hot, jnp.ones_like(x), (((0,), (0,)), ((), ())),
        preferred_element_type=jnp.float32)
    nov_ref[0, 0] += jnp.sum(1.0 - m)

    @pl.when(i == nblocks - 1)
    def _finalize():
        cnt = counts_ref[...]
        centroid = sums_ref[...] / jnp.maximum(cnt, 1.0)
        centroid = _row_normalize(centroid)
        upd = _row_normalize(DECAY * p + (1.0 - DECAY) * centroid)
        protos_out_ref[...] = jnp.where(cnt > 0.0, upd, p)
        novelty = nov_ref[0, 0] / n_tokens
        alpha = jnp.exp(la_ref[0, 0])
        sigma = jnp.exp(ls_ref[0, 0])
        gate_ref[0, 0] = 1.0 + alpha * jnp.tanh(sigma * novelty)


def _pass2_kernel(gate_ref, x_ref, out_ref):
    out_ref[...] = _gelu(x_ref[...]) * gate_ref[0, 0]


def kernel(x, protos, log_alpha, log_sigma):
    B, T, D = x.shape
    n_tokens = B * T
    x2 = x.reshape(n_tokens, D)
    block = 2048
    nblocks = n_tokens // block

    la = jnp.reshape(log_alpha, (1, 1)).astype(jnp.float32)
    ls = jnp.reshape(log_sigma, (1, 1)).astype(jnp.float32)

    new_protos, gate = pl.pallas_call(
        lambda *refs: _pass1_kernel(*refs, nblocks=nblocks, n_tokens=n_tokens),
        grid=(nblocks,),
        in_specs=[
            pl.BlockSpec((block, D), lambda i: (i, 0)),
            pl.BlockSpec((K, D), lambda i: (0, 0)),
            pl.BlockSpec(memory_space=pltpu.SMEM),
            pl.BlockSpec(memory_space=pltpu.SMEM),
        ],
        out_specs=[
            pl.BlockSpec((K, D), lambda i: (0, 0)),
            pl.BlockSpec(memory_space=pltpu.SMEM),
        ],
        out_shape=[
            jax.ShapeDtypeStruct((K, D), jnp.float32),
            jax.ShapeDtypeStruct((1, 1), jnp.float32),
        ],
        scratch_shapes=[
            pltpu.VMEM((K, D), jnp.float32),
            pltpu.VMEM((K, D), jnp.float32),
            pltpu.SMEM((1, 1), jnp.float32),
        ],
    )(x2, protos, la, ls)

    out = pl.pallas_call(
        _pass2_kernel,
        grid=(nblocks,),
        in_specs=[
            pl.BlockSpec(memory_space=pltpu.SMEM),
            pl.BlockSpec((block, D), lambda i: (i, 0)),
        ],
        out_specs=pl.BlockSpec((block, D), lambda i: (i, 0)),
        out_shape=jax.ShapeDtypeStruct((n_tokens, D), jnp.float32),
    )(gate, x2)

    return out.reshape(B, T, D), new_protos


# speedup vs baseline: 3.2294x; 1.0876x over previous
import math

import jax
import jax.numpy as jnp
from jax.experimental import pallas as pl
from jax.experimental.pallas import tpu as pltpu

K = 16
DECAY = 0.95
SQRT_2_OVER_PI = math.sqrt(2.0 / math.pi)


def _gelu(x):
    return 0.5 * x * (1.0 + jnp.tanh(SQRT_2_OVER_PI * (x + 0.044715 * x * x * x)))


def _row_normalize(v):
    n = jnp.sqrt(jnp.sum(v * v, axis=-1, keepdims=True))
    return v / jnp.maximum(n, 1e-12)


def _dot(a, b, dims, out_dtype=jnp.float32):
    return jax.lax.dot_general(a, b, (dims, ((), ())),
                               preferred_element_type=out_dtype)


def _pass1_kernel(x_ref, protos_ref, la_ref, ls_ref,
                  protos_out_ref, gate_ref,
                  sums_ref, counts_ref, nov_ref,
                  *, nblocks, n_tokens, block):
    i = pl.program_id(0)

    @pl.when(i == 0)
    def _init():
        sums_ref[...] = jnp.zeros_like(sums_ref)
        counts_ref[...] = jnp.zeros_like(counts_ref)
        nov_ref[0, 0] = 0.0

    x = x_ref[...]
    xb = x.astype(jnp.bfloat16)
    p = protos_ref[...]
    pnb = _row_normalize(p).astype(jnp.bfloat16)

    sim_t = _dot(pnb, xb, (((1,), (1,))))
    norm2 = _dot(xb * xb, jnp.ones((xb.shape[1], 1), jnp.bfloat16),
                 (((1,), (0,))))
    inv_norm = jax.lax.rsqrt(jnp.maximum(norm2, 1e-24))

    m_t = jnp.max(sim_t, axis=0, keepdims=True)
    one_hot_t = (sim_t == m_t).astype(jnp.bfloat16)

    sums_ref[...] += _dot(one_hot_t, xb, (((1,), (0,))))
    counts_ref[...] += _dot(one_hot_t,
                            jnp.ones((block, 128), jnp.bfloat16),
                            (((1,), (0,))))
    nov_ref[0, 0] += _dot(m_t, inv_norm, (((1,), (0,))))[0, 0]

    @pl.when(i == nblocks - 1)
    def _finalize():
        cnt = counts_ref[:, 0:1]
        cnt_kd = _dot(cnt, jnp.ones((1, p.shape[1]), jnp.float32),
                      (((1,), (0,))))
        centroid = sums_ref[...] / jnp.maximum(cnt_kd, 1.0)
        centroid = _row_normalize(centroid)
        upd = _row_normalize(DECAY * p + (1.0 - DECAY) * centroid)
        protos_out_ref[...] = jnp.where(cnt_kd > 0.0, upd, p)
        novelty = 1.0 - nov_ref[0, 0] / n_tokens
        alpha = jnp.exp(la_ref[0, 0])
        sigma = jnp.exp(ls_ref[0, 0])
        gate_ref[0, 0] = 1.0 + alpha * jnp.tanh(sigma * novelty)


def _pass2_kernel(gate_ref, x_ref, out_ref):
    out_ref[...] = _gelu(x_ref[...]) * gate_ref[0, 0]


def kernel(x, protos, log_alpha, log_sigma):
    B, T, D = x.shape
    n_tokens = B * T
    x2 = x.reshape(n_tokens, D)
    block = 2048
    nblocks = n_tokens // block

    la = jnp.reshape(log_alpha, (1, 1)).astype(jnp.float32)
    ls = jnp.reshape(log_sigma, (1, 1)).astype(jnp.float32)

    new_protos, gate = pl.pallas_call(
        lambda *refs: _pass1_kernel(*refs, nblocks=nblocks,
                                    n_tokens=n_tokens, block=block),
        grid=(nblocks,),
        in_specs=[
            pl.BlockSpec((block, D), lambda i: (i, 0)),
            pl.BlockSpec((K, D), lambda i: (0, 0)),
            pl.BlockSpec(memory_space=pltpu.SMEM),
            pl.BlockSpec(memory_space=pltpu.SMEM),
        ],
        out_specs=[
            pl.BlockSpec((K, D), lambda i: (0, 0)),
            pl.BlockSpec(memory_space=pltpu.SMEM),
        ],
        out_shape=[
            jax.ShapeDtypeStruct((K, D), jnp.float32),
            jax.ShapeDtypeStruct((1, 1), jnp.float32),
        ],
        scratch_shapes=[
            pltpu.VMEM((K, D), jnp.float32),
            pltpu.VMEM((K, 128), jnp.float32),
            pltpu.SMEM((1, 1), jnp.float32),
        ],
    )(x2, protos, la, ls)

    out = pl.pallas_call(
        _pass2_kernel,
        grid=(nblocks,),
        in_specs=[
            pl.BlockSpec(memory_space=pltpu.SMEM),
            pl.BlockSpec((block, D), lambda i: (i, 0)),
        ],
        out_specs=pl.BlockSpec((block, D), lambda i: (i, 0)),
        out_shape=jax.ShapeDtypeStruct((n_tokens, D), jnp.float32),
    )(gate, x2)

    return out.reshape(B, T, D), new_protos
